# baseline (speedup 1.0000x reference)
import functools

import jax
import jax.numpy as jnp
from jax import lax
from jax.experimental import pallas as pl
from jax.experimental.pallas import tpu as pltpu
from jax.experimental.pallas import tpu_sc as plsc

N = 50000
E = 800000
K_PAD = 768
HEADS = 8
PH = 8
OUT2 = 10

_f32 = jnp.float32
_i32 = jnp.int32

BLK = 1000
GRID = N // BLK

HN = N // 2
EC1 = 32
NCHUNK1 = E // EC1
C1_ITERS = 2 * (-(-NCHUNK1 // 32))
RC1 = 200
NRC1 = HN // RC1
RC1_ITERS = -(-NRC1 // 16)

EC = 128
NCHUNK = E // EC
NW = 32
CHUNK_ITERS = -(-NCHUNK // NW)
RC = 400
NRC = N // RC
RC_ITERS = -(-NRC // 16)


def _tc_a_body(x_ref, w_ref, asr_ref, adr_ref,
               t1_ref, ad_ref, ms_ref, md_ref):
    h = jnp.dot(x_ref[...], w_ref[...], preferred_element_type=_f32)
    hr = h.reshape(BLK, HEADS, PH)
    als = jnp.sum(hr * asr_ref[...][None], axis=-1)
    ald = jnp.sum(hr * adr_ref[...][None], axis=-1)
    t1_ref[...] = jnp.concatenate([h, als], axis=1)
    ad_ref[...] = ald
    bs = jnp.max(als, axis=0, keepdims=True)
    bd = jnp.max(ald, axis=0, keepdims=True)
    i = pl.program_id(0)

    @pl.when(i == 0)
    def _():
        ms_ref[...] = bs
        md_ref[...] = bd

    @pl.when(i != 0)
    def _():
        ms_ref[...] = jnp.maximum(ms_ref[...], bs)
        md_ref[...] = jnp.maximum(md_ref[...], bd)


_tc_a = pl.pallas_call(
    _tc_a_body,
    grid=(GRID,),
    in_specs=[
        pl.BlockSpec((BLK, K_PAD), lambda i: (i, 0)),
        pl.BlockSpec((K_PAD, 64), lambda i: (0, 0)),
        pl.BlockSpec((HEADS, PH), lambda i: (0, 0)),
        pl.BlockSpec((HEADS, PH), lambda i: (0, 0)),
    ],
    out_specs=[
        pl.BlockSpec((BLK, 72), lambda i: (i, 0)),
        pl.BlockSpec((BLK, 8), lambda i: (i, 0)),
        pl.BlockSpec((1, 8), lambda i: (0, 0)),
        pl.BlockSpec((1, 8), lambda i: (0, 0)),
    ],
    out_shape=[
        jax.ShapeDtypeStruct((N, 72), _f32),
        jax.ShapeDtypeStruct((N, 8), _f32),
        jax.ShapeDtypeStruct((1, 8), _f32),
        jax.ShapeDtypeStruct((1, 8), _f32),
    ],
)


def _tc_b_body(acc_ref, b1_ref, w2_ref, a2s_ref, a2d_ref,
               t2_ref, ms_ref, md_ref):
    p = acc_ref[...]
    m = p[:, :64].reshape(BLK, HEADS, PH)
    d = p[:, 64:72]
    o = (m / (d[:, :, None] + 1e-16)).reshape(BLK, 64) + b1_ref[...]
    h2 = jnp.where(o > 0, o, jnp.exp(jnp.minimum(o, 0.0)) - 1.0)
    H2 = jnp.dot(h2, w2_ref[...], preferred_element_type=_f32)
    as2 = jnp.sum(H2 * a2s_ref[...], axis=1, keepdims=True)
    ad2 = jnp.sum(H2 * a2d_ref[...], axis=1, keepdims=True)
    t2_ref[...] = jnp.concatenate(
        [H2[:, :OUT2], as2, ad2, jnp.zeros((BLK, 4), _f32)], axis=1)
    bs = jnp.full((1, 8), jnp.max(as2), _f32)
    bd = jnp.full((1, 8), jnp.max(ad2), _f32)
    i = pl.program_id(0)

    @pl.when(i == 0)
    def _():
        ms_ref[...] = bs
        md_ref[...] = bd

    @pl.when(i != 0)
    def _():
        ms_ref[...] = jnp.maximum(ms_ref[...], bs)
        md_ref[...] = jnp.maximum(md_ref[...], bd)


_tc_b = pl.pallas_call(
    _tc_b_body,
    grid=(GRID,),
    in_specs=[
        pl.BlockSpec((BLK, 72), lambda i: (i, 0)),
        pl.BlockSpec((1, 64), lambda i: (0, 0)),
        pl.BlockSpec((64, 16), lambda i: (0, 0)),
        pl.BlockSpec((1, 16), lambda i: (0, 0)),
        pl.BlockSpec((1, 16), lambda i: (0, 0)),
    ],
    out_specs=[
        pl.BlockSpec((BLK, 16), lambda i: (i, 0)),
        pl.BlockSpec((1, 8), lambda i: (0, 0)),
        pl.BlockSpec((1, 8), lambda i: (0, 0)),
    ],
    out_shape=[
        jax.ShapeDtypeStruct((N, 16), _f32),
        jax.ShapeDtypeStruct((1, 8), _f32),
        jax.ShapeDtypeStruct((1, 8), _f32),
    ],
)


def _tc_c_body(acc_ref, b2_ref, out_ref):
    a = acc_ref[...]
    s = a[0] + a[1]
    out_ref[...] = s[:, :OUT2] / (s[:, OUT2:OUT2 + 1] + 1e-16) + b2_ref[...]


_tc_c = pl.pallas_call(
    _tc_c_body,
    grid=(GRID,),
    in_specs=[
        pl.BlockSpec((2, BLK, 16), lambda i: (0, i, 0)),
        pl.BlockSpec((1, OUT2), lambda i: (0, 0)),
    ],
    out_specs=pl.BlockSpec((BLK, OUT2), lambda i: (i, 0)),
    out_shape=jax.ShapeDtypeStruct((N, OUT2), _f32),
)


@functools.cache
def _mesh():
    return plsc.VectorSubcoreMesh(core_axis_name="c", subcore_axis_name="s",
                                  num_cores=2, num_subcores=16)


def _col(v):
    return jnp.full((16,), v, _i32)


def _sc1_body(t1_hbm, ad_hbm, src_hbm, dst_hbm, cvec_hbm, z_hbm,
              acc_out,
              accS, srcv0, srcv1, dstv0, dstv1, rowsb0, rowsb1,
              advb0, advb1, dstl0, dstl1, msgv0, msgv1, cv,
              semr0, semr1, sema0, sema1, sems0, sems1):
    cid = lax.axis_index("c")
    sid = lax.axis_index("s")
    srcv = [srcv0, srcv1]
    dstv = [dstv0, dstv1]
    rows = [rowsb0, rowsb1]
    adv = [advb0, advb1]
    dstl = [dstl0, dstl1]
    msgv = [msgv0, msgv1]
    semr = [semr0, semr1]
    sema = [sema0, sema1]
    sems = [sems0, sems1]
    pltpu.sync_copy(cvec_hbm, cv)
    cval = cv[...]
    iot = jnp.arange(16, dtype=_i32)
    nbase = cid * HN

    def zero_acc(i, carry):
        r = sid + 16 * i

        @pl.when(r < NRC1)
        def _():
            pltpu.sync_copy(z_hbm, accS.at[pl.ds(r * RC1, RC1)])
        return carry

    def issue_gather(t, b):
        @pl.when(sid + 16 * t < NCHUNK1)
        def _():
            base = (sid + 16 * t) * EC1
            pltpu.sync_copy(src_hbm.at[pl.ds(base, EC1)], srcv[b])
            pltpu.sync_copy(dst_hbm.at[pl.ds(base, EC1)], dstv[b])
            pltpu.async_copy(t1_hbm.at[srcv[b]], rows[b], semr[b])
            pltpu.async_copy(ad_hbm.at[dstv[b]], adv[b], sema[b])

    def wait_gather(t, b):
        @pl.when(sid + 16 * t < NCHUNK1)
        def _():
            pltpu.make_async_copy(t1_hbm.at[srcv[b]], rows[b], semr[b]).wait()
            pltpu.make_async_copy(ad_hbm.at[dstv[b]], adv[b], sema[b]).wait()

    def compute(t, b):
        @pl.when(sid + 16 * t < NCHUNK1)
        def _():
            def grp(j, carry):
                ridx = j * 16 + iot
                dv = dstv[b][pl.ds(j * 16, 16)]
                loc = dv - nbase
                inb = (loc >= 0) & (loc < HN)
                dstl[b][pl.ds(j * 16, 16)] = jnp.where(inb, loc, HN)
                for h in range(HEADS):
                    a_s = plsc.load_gather(rows[b], [ridx, _col(64 + h)])
                    a_d = plsc.load_gather(adv[b], [ridx, _col(h)])
                    e = a_s + a_d
                    e = jnp.where(e >= 0.0, e, e * 0.2)
                    w = jnp.exp(e - cval)
                    plsc.store_scatter(msgv[b], [ridx, _col(64 + h)], w)
                    for c in range(PH):
                        col = _col(h * PH + c)
                        hv = plsc.load_gather(rows[b], [ridx, col])
                        plsc.store_scatter(msgv[b], [ridx, col], hv * w)
                return carry
            lax.fori_loop(0, EC1 // 16, grp, 0)
            pltpu.async_copy(msgv[b], accS.at[dstl[b]], add=True,
                             sem=sems[b])

    def wait_scatter(t, b):
        @pl.when((t >= 0) & (sid + 16 * t < NCHUNK1))
        def _():
            pltpu.make_async_copy(msgv[b], accS.at[dstl[b]],
                                  sems[b]).wait()

    def pipe(i, carry):
        t0 = 2 * i
        t1 = t0 + 1
        issue_gather(t1, 1)
        wait_gather(t0, 0)
        wait_scatter(t0 - 2, 0)
        compute(t0, 0)
        issue_gather(t0 + 2, 0)
        wait_gather(t1, 1)
        wait_scatter(t1 - 2, 1)
        compute(t1, 1)
        return carry

    lax.fori_loop(0, RC1_ITERS, zero_acc, 0)
    plsc.subcore_barrier()
    issue_gather(0, 0)
    lax.fori_loop(0, C1_ITERS // 2, pipe, 0)
    wait_scatter(C1_ITERS - 2, 0)
    wait_scatter(C1_ITERS - 1, 1)
    plsc.subcore_barrier()

    def dump(i, carry):
        r = sid + 16 * i

        @pl.when(r < NRC1)
        def _():
            off = nbase + r * RC1
            pltpu.sync_copy(accS.at[pl.ds(r * RC1, RC1)],
                            acc_out.at[pl.ds(off, RC1)])
        return carry
    lax.fori_loop(0, RC1_ITERS, dump, 0)


@functools.cache
def _sc1():
  return pl.kernel(
    _sc1_body,
    out_type=jax.ShapeDtypeStruct((N, 72), _f32),
    mesh=_mesh(),
    compiler_params=pltpu.CompilerParams(needs_layout_passes=False,
                                         use_tc_tiling_on_sc=False),
    scratch_types=[
        pltpu.VMEM_SHARED((HN + 8, 72), _f32),
        pltpu.VMEM((EC1,), _i32),
        pltpu.VMEM((EC1,), _i32),
        pltpu.VMEM((EC1,), _i32),
        pltpu.VMEM((EC1,), _i32),
        pltpu.VMEM((EC1, 72), _f32),
        pltpu.VMEM((EC1, 72), _f32),
        pltpu.VMEM((EC1, 8), _f32),
        pltpu.VMEM((EC1, 8), _f32),
        pltpu.VMEM((EC1,), _i32),
        pltpu.VMEM((EC1,), _i32),
        pltpu.VMEM((EC1, 72), _f32),
        pltpu.VMEM((EC1, 72), _f32),
        pltpu.VMEM((16,), _f32),
        pltpu.SemaphoreType.DMA,
        pltpu.SemaphoreType.DMA,
        pltpu.SemaphoreType.DMA,
        pltpu.SemaphoreType.DMA,
        pltpu.SemaphoreType.DMA,
        pltpu.SemaphoreType.DMA,
    ],
  )


def _sc2_body(t2_hbm, src_hbm, dst_hbm, cvec_hbm, z_hbm,
              acc_out,
              accS, srcv, dstv, rsrc, rdst, msgv, cv, sem1, sem2):
    cid = lax.axis_index("c")
    sid = lax.axis_index("s")
    wid = sid * 2 + cid
    pltpu.sync_copy(cvec_hbm, cv)
    cval = cv[...]
    iot = jnp.arange(16, dtype=_i32)
    zv = jnp.zeros((16,), _f32)
    for j in range(EC // 16):
        plsc.store_scatter(msgv, [j * 16 + iot, _col(11)], zv)

    def zero_acc(i, carry):
        r = sid + 16 * i

        @pl.when(r < NRC)
        def _():
            pltpu.sync_copy(z_hbm, accS.at[pl.ds(r * RC, RC)])
        return carry

    def chunk(i, carry):
        k = wid + NW * i

        @pl.when(k < NCHUNK)
        def _():
            base = k * EC
            hs = pltpu.async_copy(src_hbm.at[pl.ds(base, EC)], srcv, sem1)
            hd = pltpu.async_copy(dst_hbm.at[pl.ds(base, EC)], dstv, sem2)
            hs.wait()
            hd.wait()
            h1 = pltpu.async_copy(t2_hbm.at[srcv], rsrc, sem1)
            h2 = pltpu.async_copy(t2_hbm.at[dstv], rdst, sem2)
            h1.wait()
            h2.wait()
            for j in range(EC // 16):
                ridx = j * 16 + iot
                a_s = plsc.load_gather(rsrc, [ridx, _col(10)])
                a_d = plsc.load_gather(rdst, [ridx, _col(11)])
                e = a_s + a_d
                e = jnp.where(e >= 0.0, e, e * 0.2)
                w = jnp.exp(e - cval)
                plsc.store_scatter(msgv, [ridx, _col(10)], w)
                for c in range(OUT2):
                    hv = plsc.load_gather(rsrc, [ridx, _col(c)])
                    plsc.store_scatter(msgv, [ridx, _col(c)], hv * w)
            pltpu.sync_copy(msgv, accS.at[dstv], add=True)
        return carry

    lax.fori_loop(0, RC_ITERS, zero_acc, 0)
    plsc.subcore_barrier()
    lax.fori_loop(0, CHUNK_ITERS, chunk, 0)
    plsc.subcore_barrier()

    def dump(i, carry):
        r = sid + 16 * i

        @pl.when(r < NRC)
        def _():
            off = cid * N + r * RC
            pltpu.sync_copy(accS.at[pl.ds(r * RC, RC)],
                            acc_out.at[pl.ds(off, RC)])
        return carry
    lax.fori_loop(0, RC_ITERS, dump, 0)


@functools.cache
def _sc2():
  return pl.kernel(
    _sc2_body,
    out_type=jax.ShapeDtypeStruct((2 * N, 16), _f32),
    mesh=_mesh(),
    compiler_params=pltpu.CompilerParams(needs_layout_passes=False,
                                         use_tc_tiling_on_sc=False),
    scratch_types=[
        pltpu.VMEM_SHARED((N, 16), _f32),
        pltpu.VMEM((EC,), _i32),
        pltpu.VMEM((EC,), _i32),
        pltpu.VMEM((EC, 16), _f32),
        pltpu.VMEM((EC, 16), _f32),
        pltpu.VMEM((EC, 16), _f32),
        pltpu.VMEM((16,), _f32),
        pltpu.SemaphoreType.DMA,
        pltpu.SemaphoreType.DMA,
    ],
  )


@jax.jit
def kernel(x, edge_index, W1, att_src1, att_dst1, b1,
           W2, att_src2, att_dst2, b2):
    x_p = jnp.pad(x, ((0, 0), (0, K_PAD - x.shape[1])))
    W1_p = jnp.pad(W1, ((0, K_PAD - W1.shape[0]), (0, 0)))
    asr = att_src1.reshape(HEADS, PH)
    adr = att_dst1.reshape(HEADS, PH)
    t1, ad, ms, md = _tc_a(x_p, W1_p, asr, adr)
    c1 = jnp.maximum(jnp.max(ms) + jnp.max(md), 0.0)
    c1v = jnp.full((16,), c1, _f32)
    src = edge_index[0]
    dst = edge_index[1]
    z1 = jnp.zeros((RC1, 72), _f32)
    acc1 = _sc1()(t1, ad, src, dst, c1v, z1)
    W2_p = jnp.pad(W2, ((0, 0), (0, 6)))
    a2s = jnp.pad(att_src2.reshape(1, OUT2), ((0, 0), (0, 6)))
    a2d = jnp.pad(att_dst2.reshape(1, OUT2), ((0, 0), (0, 6)))
    t2, ms2, md2 = _tc_b(acc1, b1.reshape(1, 64), W2_p, a2s, a2d)
    c2 = jnp.maximum(jnp.max(ms2) + jnp.max(md2), 0.0)
    c2v = jnp.full((16,), c2, _f32)
    z2 = jnp.zeros((RC, 16), _f32)
    acc2 = _sc2()(t2, src, dst, c2v, z2)
    out = _tc_c(acc2.reshape(2, N, 16), b2.reshape(1, OUT2))
    return out


# speedup vs baseline: 1.2243x; 1.2243x over previous
import functools

import jax
import jax.numpy as jnp
from jax import lax
from jax.experimental import pallas as pl
from jax.experimental.pallas import tpu as pltpu
from jax.experimental.pallas import tpu_sc as plsc

N = 50000
E = 800000
K_PAD = 768
HEADS = 8
PH = 8
OUT2 = 10

_f32 = jnp.float32
_i32 = jnp.int32

BLK = 1000
GRID = N // BLK

W1R = 40
EC1 = 32
NCHUNK1 = E // EC1
C1_ITERS = 2 * (-(-NCHUNK1 // 32))
RC1 = 500
NRC1 = N // RC1
RC1_ITERS = -(-NRC1 // 16)

EC = 128
NCHUNK = E // EC
NW = 32
CHUNK_ITERS = -(-NCHUNK // NW)
RC = 400
NRC = N // RC
RC_ITERS = -(-NRC // 16)


def _tc_a_body(x_ref, w_ref, asr_ref, adr_ref,
               t1_ref, ad_ref, ms_ref, md_ref):
    h = jnp.dot(x_ref[...], w_ref[...], preferred_element_type=_f32)
    hr = h.reshape(BLK, HEADS, PH)
    als = jnp.sum(hr * asr_ref[...][None], axis=-1)
    ald = jnp.sum(hr * adr_ref[...][None], axis=-1)
    pad = jnp.zeros((BLK, 4), _f32)
    p0 = jnp.concatenate([h[:, :32], als[:, :4], pad], axis=1)
    p1 = jnp.concatenate([h[:, 32:], als[:, 4:], pad], axis=1)
    t1_ref[...] = jnp.stack([p0, p1], axis=0)
    ad_ref[...] = ald
    bs = jnp.max(als, axis=0, keepdims=True)
    bd = jnp.max(ald, axis=0, keepdims=True)
    i = pl.program_id(0)

    @pl.when(i == 0)
    def _():
        ms_ref[...] = bs
        md_ref[...] = bd

    @pl.when(i != 0)
    def _():
        ms_ref[...] = jnp.maximum(ms_ref[...], bs)
        md_ref[...] = jnp.maximum(md_ref[...], bd)


_tc_a = pl.pallas_call(
    _tc_a_body,
    grid=(GRID,),
    in_specs=[
        pl.BlockSpec((BLK, K_PAD), lambda i: (i, 0)),
        pl.BlockSpec((K_PAD, 64), lambda i: (0, 0)),
        pl.BlockSpec((HEADS, PH), lambda i: (0, 0)),
        pl.BlockSpec((HEADS, PH), lambda i: (0, 0)),
    ],
    out_specs=[
        pl.BlockSpec((2, BLK, W1R), lambda i: (0, i, 0)),
        pl.BlockSpec((BLK, 8), lambda i: (i, 0)),
        pl.BlockSpec((1, 8), lambda i: (0, 0)),
        pl.BlockSpec((1, 8), lambda i: (0, 0)),
    ],
    out_shape=[
        jax.ShapeDtypeStruct((2, N, W1R), _f32),
        jax.ShapeDtypeStruct((N, 8), _f32),
        jax.ShapeDtypeStruct((1, 8), _f32),
        jax.ShapeDtypeStruct((1, 8), _f32),
    ],
)


def _tc_b_body(acc_ref, b1_ref, w2_ref, a2s_ref, a2d_ref,
               t2_ref, ms_ref, md_ref):
    p = acc_ref[...]
    m = jnp.concatenate([p[0, :, :32], p[1, :, :32]],
                        axis=1).reshape(BLK, HEADS, PH)
    d = jnp.concatenate([p[0, :, 32:36], p[1, :, 32:36]], axis=1)
    o = (m / (d[:, :, None] + 1e-16)).reshape(BLK, 64) + b1_ref[...]
    h2 = jnp.where(o > 0, o, jnp.exp(jnp.minimum(o, 0.0)) - 1.0)
    H2 = jnp.dot(h2, w2_ref[...], preferred_element_type=_f32)
    as2 = jnp.sum(H2 * a2s_ref[...], axis=1, keepdims=True)
    ad2 = jnp.sum(H2 * a2d_ref[...], axis=1, keepdims=True)
    t2_ref[...] = jnp.concatenate(
        [H2[:, :OUT2], as2, ad2, jnp.zeros((BLK, 4), _f32)], axis=1)
    bs = jnp.full((1, 8), jnp.max(as2), _f32)
    bd = jnp.full((1, 8), jnp.max(ad2), _f32)
    i = pl.program_id(0)

    @pl.when(i == 0)
    def _():
        ms_ref[...] = bs
        md_ref[...] = bd

    @pl.when(i != 0)
    def _():
        ms_ref[...] = jnp.maximum(ms_ref[...], bs)
        md_ref[...] = jnp.maximum(md_ref[...], bd)


_tc_b = pl.pallas_call(
    _tc_b_body,
    grid=(GRID,),
    in_specs=[
        pl.BlockSpec((2, BLK, W1R), lambda i: (0, i, 0)),
        pl.BlockSpec((1, 64), lambda i: (0, 0)),
        pl.BlockSpec((64, 16), lambda i: (0, 0)),
        pl.BlockSpec((1, 16), lambda i: (0, 0)),
        pl.BlockSpec((1, 16), lambda i: (0, 0)),
    ],
    out_specs=[
        pl.BlockSpec((BLK, 16), lambda i: (i, 0)),
        pl.BlockSpec((1, 8), lambda i: (0, 0)),
        pl.BlockSpec((1, 8), lambda i: (0, 0)),
    ],
    out_shape=[
        jax.ShapeDtypeStruct((N, 16), _f32),
        jax.ShapeDtypeStruct((1, 8), _f32),
        jax.ShapeDtypeStruct((1, 8), _f32),
    ],
)


def _tc_c_body(acc_ref, b2_ref, out_ref):
    a = acc_ref[...]
    s = a[0] + a[1]
    out_ref[...] = s[:, :OUT2] / (s[:, OUT2:OUT2 + 1] + 1e-16) + b2_ref[...]


_tc_c = pl.pallas_call(
    _tc_c_body,
    grid=(GRID,),
    in_specs=[
        pl.BlockSpec((2, BLK, 16), lambda i: (0, i, 0)),
        pl.BlockSpec((1, OUT2), lambda i: (0, 0)),
    ],
    out_specs=pl.BlockSpec((BLK, OUT2), lambda i: (i, 0)),
    out_shape=jax.ShapeDtypeStruct((N, OUT2), _f32),
)


@functools.cache
def _mesh():
    return plsc.VectorSubcoreMesh(core_axis_name="c", subcore_axis_name="s",
                                  num_cores=2, num_subcores=16)


def _col(v):
    return jnp.full((16,), v, _i32)


def _sc1_body(t1_hbm, ad_hbm, src_hbm, dst_hbm, cvec_hbm, z_hbm,
              acc_out,
              accS, srcv0, srcv1, dstv0, dstv1, rowsb0, rowsb1,
              advb0, advb1, msgv0, msgv1, cv,
              semr0, semr1, sema0, sema1, sems0, sems1):
    cid = lax.axis_index("c")
    sid = lax.axis_index("s")
    srcv = [srcv0, srcv1]
    dstv = [dstv0, dstv1]
    rows = [rowsb0, rowsb1]
    adv = [advb0, advb1]
    msgv = [msgv0, msgv1]
    semr = [semr0, semr1]
    sema = [sema0, sema1]
    sems = [sems0, sems1]
    pltpu.sync_copy(cvec_hbm, cv)
    cval = cv[...]
    iot = jnp.arange(16, dtype=_i32)
    hbase = cid * N
    hcol = 4 * cid
    zv = jnp.zeros((16,), _f32)
    for b in range(2):
        for j in range(EC1 // 16):
            for c in range(36, W1R):
                plsc.store_scatter(msgv[b], [j * 16 + iot, _col(c)], zv)

    def zero_acc(i, carry):
        r = sid + 16 * i

        @pl.when(r < NRC1)
        def _():
            pltpu.sync_copy(z_hbm, accS.at[pl.ds(r * RC1, RC1)])
        return carry

    def issue_gather(t, b):
        @pl.when(sid + 16 * t < NCHUNK1)
        def _():
            base = (sid + 16 * t) * EC1
            pltpu.sync_copy(src_hbm.at[pl.ds(base, EC1)], srcv[b])
            pltpu.sync_copy(dst_hbm.at[pl.ds(base, EC1)], dstv[b])
            for j in range(EC1 // 16):
                sl = pl.ds(j * 16, 16)
                srcv[b][sl] = srcv[b][sl] + hbase
            pltpu.async_copy(t1_hbm.at[srcv[b]], rows[b], semr[b])
            pltpu.async_copy(ad_hbm.at[dstv[b]], adv[b], sema[b])

    def wait_gather(t, b):
        @pl.when(sid + 16 * t < NCHUNK1)
        def _():
            pltpu.make_async_copy(t1_hbm.at[srcv[b]], rows[b], semr[b]).wait()
            pltpu.make_async_copy(ad_hbm.at[dstv[b]], adv[b], sema[b]).wait()

    def compute(t, b):
        @pl.when(sid + 16 * t < NCHUNK1)
        def _():
            def grp(j, carry):
                ridx = j * 16 + iot
                for h in range(4):
                    a_s = plsc.load_gather(rows[b], [ridx, _col(32 + h)])
                    a_d = plsc.load_gather(adv[b], [ridx, _col(hcol + h)])
                    e = a_s + a_d
                    e = jnp.where(e >= 0.0, e, e * 0.2)
                    w = jnp.exp(e - cval)
                    plsc.store_scatter(msgv[b], [ridx, _col(32 + h)], w)
                    for c in range(PH):
                        col = _col(h * PH + c)
                        hv = plsc.load_gather(rows[b], [ridx, col])
                        plsc.store_scatter(msgv[b], [ridx, col], hv * w)
                return carry
            lax.fori_loop(0, EC1 // 16, grp, 0)
            pltpu.async_copy(msgv[b], accS.at[dstv[b]], add=True,
                             sem=sems[b])

    def wait_scatter(t, b):
        @pl.when((t >= 0) & (sid + 16 * t < NCHUNK1))
        def _():
            pltpu.make_async_copy(msgv[b], accS.at[dstv[b]],
                                  sems[b]).wait()

    def pipe(i, carry):
        t0 = 2 * i
        t1 = t0 + 1
        issue_gather(t1, 1)
        wait_gather(t0, 0)
        wait_scatter(t0 - 2, 0)
        compute(t0, 0)
        issue_gather(t0 + 2, 0)
        wait_gather(t1, 1)
        wait_scatter(t1 - 2, 1)
        compute(t1, 1)
        return carry

    lax.fori_loop(0, RC1_ITERS, zero_acc, 0)
    plsc.subcore_barrier()
    issue_gather(0, 0)
    lax.fori_loop(0, C1_ITERS // 2, pipe, 0)
    wait_scatter(C1_ITERS - 2, 0)
    wait_scatter(C1_ITERS - 1, 1)
    plsc.subcore_barrier()

    def dump(i, carry):
        r = sid + 16 * i

        @pl.when(r < NRC1)
        def _():
            off = cid * N + r * RC1
            pltpu.sync_copy(accS.at[pl.ds(r * RC1, RC1)],
                            acc_out.at[pl.ds(off, RC1)])
        return carry
    lax.fori_loop(0, RC1_ITERS, dump, 0)


@functools.cache
def _sc1():
  return pl.kernel(
    _sc1_body,
    out_type=jax.ShapeDtypeStruct((2 * N, W1R), _f32),
    mesh=_mesh(),
    compiler_params=pltpu.CompilerParams(needs_layout_passes=False,
                                         use_tc_tiling_on_sc=False),
    scratch_types=[
        pltpu.VMEM_SHARED((N, W1R), _f32),
        pltpu.VMEM((EC1,), _i32),
        pltpu.VMEM((EC1,), _i32),
        pltpu.VMEM((EC1,), _i32),
        pltpu.VMEM((EC1,), _i32),
        pltpu.VMEM((EC1, W1R), _f32),
        pltpu.VMEM((EC1, W1R), _f32),
        pltpu.VMEM((EC1, 8), _f32),
        pltpu.VMEM((EC1, 8), _f32),
        pltpu.VMEM((EC1, W1R), _f32),
        pltpu.VMEM((EC1, W1R), _f32),
        pltpu.VMEM((16,), _f32),
        pltpu.SemaphoreType.DMA,
        pltpu.SemaphoreType.DMA,
        pltpu.SemaphoreType.DMA,
        pltpu.SemaphoreType.DMA,
        pltpu.SemaphoreType.DMA,
        pltpu.SemaphoreType.DMA,
    ],
  )


def _sc2_body(t2_hbm, src_hbm, dst_hbm, cvec_hbm, z_hbm,
              acc_out,
              accS, srcv, dstv, rsrc, rdst, msgv, cv, sem1, sem2):
    cid = lax.axis_index("c")
    sid = lax.axis_index("s")
    wid = sid * 2 + cid
    pltpu.sync_copy(cvec_hbm, cv)
    cval = cv[...]
    iot = jnp.arange(16, dtype=_i32)
    zv = jnp.zeros((16,), _f32)
    for j in range(EC // 16):
        plsc.store_scatter(msgv, [j * 16 + iot, _col(11)], zv)

    def zero_acc(i, carry):
        r = sid + 16 * i

        @pl.when(r < NRC)
        def _():
            pltpu.sync_copy(z_hbm, accS.at[pl.ds(r * RC, RC)])
        return carry

    def chunk(i, carry):
        k = wid + NW * i

        @pl.when(k < NCHUNK)
        def _():
            base = k * EC
            hs = pltpu.async_copy(src_hbm.at[pl.ds(base, EC)], srcv, sem1)
            hd = pltpu.async_copy(dst_hbm.at[pl.ds(base, EC)], dstv, sem2)
            hs.wait()
            hd.wait()
            h1 = pltpu.async_copy(t2_hbm.at[srcv], rsrc, sem1)
            h2 = pltpu.async_copy(t2_hbm.at[dstv], rdst, sem2)
            h1.wait()
            h2.wait()
            for j in range(EC // 16):
                ridx = j * 16 + iot
                a_s = plsc.load_gather(rsrc, [ridx, _col(10)])
                a_d = plsc.load_gather(rdst, [ridx, _col(11)])
                e = a_s + a_d
                e = jnp.where(e >= 0.0, e, e * 0.2)
                w = jnp.exp(e - cval)
                plsc.store_scatter(msgv, [ridx, _col(10)], w)
                for c in range(OUT2):
                    hv = plsc.load_gather(rsrc, [ridx, _col(c)])
                    plsc.store_scatter(msgv, [ridx, _col(c)], hv * w)
            pltpu.sync_copy(msgv, accS.at[dstv], add=True)
        return carry

    lax.fori_loop(0, RC_ITERS, zero_acc, 0)
    plsc.subcore_barrier()
    lax.fori_loop(0, CHUNK_ITERS, chunk, 0)
    plsc.subcore_barrier()

    def dump(i, carry):
        r = sid + 16 * i

        @pl.when(r < NRC)
        def _():
            off = cid * N + r * RC
            pltpu.sync_copy(accS.at[pl.ds(r * RC, RC)],
                            acc_out.at[pl.ds(off, RC)])
        return carry
    lax.fori_loop(0, RC_ITERS, dump, 0)


@functools.cache
def _sc2():
  return pl.kernel(
    _sc2_body,
    out_type=jax.ShapeDtypeStruct((2 * N, 16), _f32),
    mesh=_mesh(),
    compiler_params=pltpu.CompilerParams(needs_layout_passes=False,
                                         use_tc_tiling_on_sc=False),
    scratch_types=[
        pltpu.VMEM_SHARED((N, 16), _f32),
        pltpu.VMEM((EC,), _i32),
        pltpu.VMEM((EC,), _i32),
        pltpu.VMEM((EC, 16), _f32),
        pltpu.VMEM((EC, 16), _f32),
        pltpu.VMEM((EC, 16), _f32),
        pltpu.VMEM((16,), _f32),
        pltpu.SemaphoreType.DMA,
        pltpu.SemaphoreType.DMA,
    ],
  )


@jax.jit
def kernel(x, edge_index, W1, att_src1, att_dst1, b1,
           W2, att_src2, att_dst2, b2):
    x_p = jnp.pad(x, ((0, 0), (0, K_PAD - x.shape[1])))
    W1_p = jnp.pad(W1, ((0, K_PAD - W1.shape[0]), (0, 0)))
    asr = att_src1.reshape(HEADS, PH)
    adr = att_dst1.reshape(HEADS, PH)
    t1, ad, ms, md = _tc_a(x_p, W1_p, asr, adr)
    c1 = jnp.maximum(jnp.max(ms) + jnp.max(md), 0.0)
    c1v = jnp.full((16,), c1, _f32)
    src = edge_index[0]
    dst = edge_index[1]
    z1 = jnp.zeros((RC1, W1R), _f32)
    acc1 = _sc1()(t1.reshape(2 * N, W1R), ad, src, dst, c1v, z1)
    W2_p = jnp.pad(W2, ((0, 0), (0, 6)))
    a2s = jnp.pad(att_src2.reshape(1, OUT2), ((0, 0), (0, 6)))
    a2d = jnp.pad(att_dst2.reshape(1, OUT2), ((0, 0), (0, 6)))
    t2, ms2, md2 = _tc_b(acc1.reshape(2, N, W1R),
                         b1.reshape(1, 64), W2_p, a2s, a2d)
    c2 = jnp.maximum(jnp.max(ms2) + jnp.max(md2), 0.0)
    c2v = jnp.full((16,), c2, _f32)
    z2 = jnp.zeros((RC, 16), _f32)
    acc2 = _sc2()(t2, src, dst, c2v, z2)
    out = _tc_c(acc2.reshape(2, N, 16), b2.reshape(1, OUT2))
    return out
